# baseline (speedup 1.0000x reference)
import functools

import jax
import jax.numpy as jnp
from jax import lax
from jax.experimental import pallas as pl
from jax.experimental.pallas import tpu as pltpu
from jax.experimental.pallas import tpu_sc as plsc

VOCAB = 23236
EMBED_DIM = 1728
BATCH = 64
SEQ = 512
N = BATCH * SEQ

NC = 2
NS = 16
NW = NC * NS
PER_W = N // NW
CHUNK = 32
NCHUNK = PER_W // CHUNK


def _gather_body(idx_hbm, table_hbm, out_hbm, idx_v, buf0, buf1, sem0, sem1):
    wid = lax.axis_index("s") * NC + lax.axis_index("c")
    base = wid * PER_W
    pltpu.sync_copy(idx_hbm.at[wid], idx_v)
    pltpu.make_async_copy(table_hbm.at[idx_v.at[0]], buf0, sem0).start()

    bufs = (buf0, buf1)
    sems = (sem0, sem1)

    @pl.loop(0, NCHUNK, step=2)
    def _(c0):
        for b in range(2):
            c = c0 + b
            buf, sem = bufs[b], sems[b]
            nbuf, nsem = bufs[1 - b], sems[1 - b]
            pltpu.make_async_copy(table_hbm.at[idx_v.at[c]], buf, sem).wait()

            @pl.when(c + 1 < NCHUNK)
            def _():
                pltpu.make_async_copy(
                    table_hbm.at[idx_v.at[c + 1]], nbuf, nsem
                ).start()

            pltpu.sync_copy(buf, out_hbm.at[pl.ds(base + c * CHUNK, CHUNK)])


_gather = pl.kernel(
    _gather_body,
    out_type=jax.ShapeDtypeStruct((N, EMBED_DIM), jnp.float32),
    mesh=plsc.VectorSubcoreMesh(core_axis_name="c", subcore_axis_name="s"),
    scratch_types=[
        pltpu.VMEM((NCHUNK, CHUNK), jnp.int32),
        pltpu.VMEM((CHUNK, EMBED_DIM), jnp.float32),
        pltpu.VMEM((CHUNK, EMBED_DIM), jnp.float32),
        pltpu.SemaphoreType.DMA,
        pltpu.SemaphoreType.DMA,
    ],
    compiler_params=pltpu.CompilerParams(use_tc_tiling_on_sc=False),
)


@jax.jit
def kernel(inputs, table):
    idx = inputs.astype(jnp.int32).reshape(NW, NCHUNK, CHUNK)
    out = _gather(idx, table)
    return out.reshape(BATCH, SEQ, EMBED_DIM)


# speedup vs baseline: 1.0013x; 1.0013x over previous
import functools

import jax
import jax.numpy as jnp
from jax import lax
from jax.experimental import pallas as pl
from jax.experimental.pallas import tpu as pltpu
from jax.experimental.pallas import tpu_sc as plsc

VOCAB = 23236
EMBED_DIM = 1728
BATCH = 64
SEQ = 512
N = BATCH * SEQ

NC = 2
NS = 16
NW = NC * NS
PER_W = N // NW
CHUNK = 16
NCHUNK = PER_W // CHUNK
NB = 4


def _gather_body(idx_hbm, table_hbm, out_hbm, idx_v, *rest):
    bufs = rest[:NB]
    gsems = rest[NB : 2 * NB]
    wsems = rest[2 * NB : 3 * NB]
    wid = lax.axis_index("s") * NC + lax.axis_index("c")
    base = wid * PER_W
    pltpu.sync_copy(idx_hbm.at[wid], idx_v)
    for b in range(NB - 1):
        pltpu.make_async_copy(table_hbm.at[idx_v.at[b]], bufs[b], gsems[b]).start()

    @pl.loop(0, NCHUNK, step=NB)
    def _(c0):
        for b in range(NB):
            c = c0 + b
            buf, gsem, wsem = bufs[b], gsems[b], wsems[b]
            pltpu.make_async_copy(table_hbm.at[idx_v.at[c]], buf, gsem).wait()
            out_slc = out_hbm.at[pl.ds(base + c * CHUNK, CHUNK)]
            pltpu.make_async_copy(buf, out_slc, wsem).start()
            pb = (b + NB - 1) % NB
            pc = c - 1

            @pl.when(c + NB - 1 < NCHUNK)
            def _():
                pslc = out_hbm.at[pl.ds(base + pc * CHUNK, CHUNK)]

                @pl.when(c >= 1)
                def _():
                    pltpu.make_async_copy(bufs[pb], pslc, wsems[pb]).wait()

                pltpu.make_async_copy(
                    table_hbm.at[idx_v.at[c + NB - 1]], bufs[pb], gsems[pb]
                ).start()

    for k in range(NB, 0, -1):
        c = NCHUNK - k
        b = c % NB
        out_slc = out_hbm.at[pl.ds(base + c * CHUNK, CHUNK)]
        pltpu.make_async_copy(bufs[b], out_slc, wsems[b]).wait()


_gather = pl.kernel(
    _gather_body,
    out_type=jax.ShapeDtypeStruct((N, EMBED_DIM), jnp.float32),
    mesh=plsc.VectorSubcoreMesh(core_axis_name="c", subcore_axis_name="s"),
    scratch_types=[
        pltpu.VMEM((NCHUNK, CHUNK), jnp.int32),
        *[pltpu.VMEM((CHUNK, EMBED_DIM), jnp.float32) for _ in range(NB)],
        *[pltpu.SemaphoreType.DMA for _ in range(2 * NB)],
    ],
    compiler_params=pltpu.CompilerParams(use_tc_tiling_on_sc=False),
)


@jax.jit
def kernel(inputs, table):
    idx = inputs.astype(jnp.int32).reshape(NW, NCHUNK, CHUNK)
    out = _gather(idx, table)
    return out.reshape(BATCH, SEQ, EMBED_DIM)
